# baseline (speedup 1.0000x reference)
import functools

import jax
import jax.numpy as jnp
from jax.experimental import pallas as pl
from jax.experimental.pallas import tpu as pltpu

_VMEM_LIMIT = 48 * 1024 * 1024


def _scse_kernel(x_ref, sel_ref, w1t_ref, b1_ref, w2t_ref, b2_ref, ws_ref,
                 bs_ref, o_ref, *, hw, imgs, inv_hw):
    xf = x_ref[...]
    xb = xf.reshape(imgs * hw, xf.shape[2]).astype(jnp.bfloat16)

    sp = jnp.dot(xb, ws_ref[...],
                 preferred_element_type=jnp.float32) + bs_ref[0]
    sse = jax.nn.sigmoid(sp).reshape(imgs, hw, 1)

    pools = jnp.dot(sel_ref[...], xb,
                    preferred_element_type=jnp.float32) * inv_hw
    z = jnp.dot(pools, w1t_ref[...],
                preferred_element_type=jnp.float32) + b1_ref[...]
    z = jnp.maximum(z, 0.0)
    s = jnp.dot(z, w2t_ref[...],
                preferred_element_type=jnp.float32) + b2_ref[...]
    cse = jax.nn.sigmoid(s)[:, None, :]

    o_ref[...] = xf * (cse + sse)


def kernel(x, w1, b1, w2, b2, ws, bs):
    N, C, H, W = x.shape
    HW = H * W
    mid = w1.shape[0]
    B = 2

    xt = jnp.transpose(x, (0, 2, 3, 1)).reshape(N, HW, C)

    sel = jnp.repeat(jnp.eye(B, dtype=jnp.bfloat16), HW, axis=1)
    w1t = w1.astype(jnp.float32).T
    w2t = w2.astype(jnp.float32).T
    b1r = b1.reshape(1, mid).astype(jnp.float32)
    b2r = b2.reshape(1, C).astype(jnp.float32)
    ws_col = ws.reshape(1, C).T.astype(jnp.bfloat16)
    bs_smem = bs.reshape(1).astype(jnp.float32)

    out = pl.pallas_call(
        functools.partial(_scse_kernel, hw=HW, imgs=B, inv_hw=1.0 / HW),
        out_shape=jax.ShapeDtypeStruct((N, HW, C), jnp.float32),
        grid_spec=pltpu.PrefetchScalarGridSpec(
            num_scalar_prefetch=0,
            grid=(N // B,),
            in_specs=[
                pl.BlockSpec((B, HW, C), lambda n: (n, 0, 0)),
                pl.BlockSpec((B, B * HW), lambda n: (0, 0)),
                pl.BlockSpec((C, mid), lambda n: (0, 0)),
                pl.BlockSpec((1, mid), lambda n: (0, 0)),
                pl.BlockSpec((mid, C), lambda n: (0, 0)),
                pl.BlockSpec((1, C), lambda n: (0, 0)),
                pl.BlockSpec((C, 1), lambda n: (0, 0)),
                pl.BlockSpec(memory_space=pltpu.MemorySpace.SMEM),
            ],
            out_specs=pl.BlockSpec((B, HW, C), lambda n: (n, 0, 0)),
        ),
        compiler_params=pltpu.CompilerParams(
            dimension_semantics=("parallel",),
            vmem_limit_bytes=_VMEM_LIMIT),
    )(xt, sel, w1t, b1r, w2t, b2r, ws_col, bs_smem)

    return jnp.transpose(out.reshape(N, H, W, C), (0, 3, 1, 2))
